# baseline (speedup 1.0000x reference)
import jax
import jax.numpy as jnp
from jax import lax
from jax.experimental import pallas as pl
from jax.experimental.pallas import tpu as pltpu
from jax.experimental.pallas import tpu_sc as plsc

NC = 2
NS = 16
NW = NC * NS

D = 128
CHUNK = 128
NBUF = 5


def _gather_body(x_hbm, w_hbm, out_hbm, tbl_s, idx_v, rows_v, gsem, ssem):
    sid = lax.axis_index("s")
    wid = sid * NC + lax.axis_index("c")
    n_chunks = x_hbm.shape[1]
    base = wid * (n_chunks * CHUNK)

    @pl.when(sid == 0)
    def _():
        pltpu.sync_copy(w_hbm, tbl_s)

    pltpu.sync_copy(x_hbm.at[wid], idx_v)
    plsc.subcore_barrier()

    def fire_gather(j, b):
        src = tbl_s if b % 2 == 0 else w_hbm
        pltpu.async_copy(src.at[idx_v.at[j]], rows_v.at[b], gsem.at[b])

    def fire_scatter(j, b):
        pltpu.async_copy(
            rows_v.at[b], out_hbm.at[pl.ds(base + j * CHUNK, CHUNK)], ssem.at[b]
        )

    def wait_gather(b):
        pltpu.make_async_copy(
            out_hbm.at[pl.ds(base, CHUNK)], rows_v.at[b], gsem.at[b]
        ).wait()

    def wait_scatter(j, b):
        pltpu.make_async_copy(
            rows_v.at[b], out_hbm.at[pl.ds(base + j * CHUNK, CHUNK)], ssem.at[b]
        ).wait()

    for b in range(NBUF):
        fire_gather(b, b)

    def body(o, _):
        j0 = o * NBUF
        for b in range(NBUF):
            j = j0 + b
            wait_gather(b)
            fire_scatter(j, b)
            wait_scatter(j, b)
            fire_gather(j + NBUF, b)
        return 0

    lax.fori_loop(0, (n_chunks - NBUF) // NBUF, body, 0, unroll=False)

    for b in range(NBUF):
        j = n_chunks - NBUF + b
        wait_gather(b)
        fire_scatter(j, b)
    for b in range(NBUF):
        wait_scatter(n_chunks - NBUF + b, b)


def kernel(x, weight):
    B, S = x.shape
    total = B * S
    per_w = total // NW
    n_chunks = per_w // CHUNK
    x3 = x.reshape(NW, n_chunks, CHUNK)

    mesh = plsc.VectorSubcoreMesh(
        core_axis_name="c", subcore_axis_name="s", num_cores=NC, num_subcores=NS
    )
    out = pl.kernel(
        _gather_body,
        out_type=jax.ShapeDtypeStruct((total, D), jnp.float32),
        mesh=mesh,
        scratch_types=[
            pltpu.VMEM_SHARED(weight.shape, jnp.float32),
            pltpu.VMEM((n_chunks, CHUNK), jnp.int32),
            pltpu.VMEM((NBUF, CHUNK, D), jnp.float32),
            pltpu.SemaphoreType.DMA((NBUF,)),
            pltpu.SemaphoreType.DMA((NBUF,)),
        ],
    )(x3, weight)
    return lax.stop_gradient(out.reshape(B, S, D))


# speedup vs baseline: 1.7180x; 1.7180x over previous
import jax
import jax.numpy as jnp
from jax import lax
from jax.experimental import pallas as pl
from jax.experimental.pallas import tpu as pltpu
from jax.experimental.pallas import tpu_sc as plsc

NC = 2
NS = 16
NW = NC * NS

D = 128
CHUNK = 128
GROUP = 2
ROWS = GROUP * CHUNK
NBUF = 2


def _gather_body(x_hbm, w_hbm, out_hbm, tbl_s, idx_v, rows_v, gsem, ssem):
    sid = lax.axis_index("s")
    wid = sid * NC + lax.axis_index("c")
    n_groups = x_hbm.shape[1] // GROUP
    base = wid * (n_groups * ROWS)

    @pl.when(sid == 0)
    def _():
        pltpu.sync_copy(w_hbm, tbl_s)

    pltpu.sync_copy(x_hbm.at[wid], idx_v)
    plsc.subcore_barrier()

    def fire_gather(j, b):
        for h in range(GROUP):
            pltpu.async_copy(
                tbl_s.at[idx_v.at[GROUP * j + h]],
                rows_v.at[b].at[pl.ds(h * CHUNK, CHUNK)],
                gsem.at[b],
            )

    def fire_scatter(j, b):
        pltpu.async_copy(
            rows_v.at[b], out_hbm.at[pl.ds(base + j * ROWS, ROWS)], ssem.at[b]
        )

    def wait_gather(b):
        pltpu.make_async_copy(
            out_hbm.at[pl.ds(base, ROWS)], rows_v.at[b], gsem.at[b]
        ).wait()

    def wait_scatter(j, b):
        pltpu.make_async_copy(
            rows_v.at[b], out_hbm.at[pl.ds(base + j * ROWS, ROWS)], ssem.at[b]
        ).wait()

    for b in range(NBUF):
        fire_gather(b, b)

    def body(o, _):
        j0 = o * NBUF
        for b in range(NBUF):
            j = j0 + b
            wait_gather(b)
            fire_scatter(j, b)
            wait_scatter(j, b)
            fire_gather(j + NBUF, b)
        return 0

    lax.fori_loop(0, (n_groups - NBUF) // NBUF, body, 0, unroll=False)

    for b in range(NBUF):
        j = n_groups - NBUF + b
        wait_gather(b)
        fire_scatter(j, b)
    for b in range(NBUF):
        wait_scatter(n_groups - NBUF + b, b)


def kernel(x, weight):
    B, S = x.shape
    total = B * S
    per_w = total // NW
    n_chunks = per_w // CHUNK
    x3 = x.reshape(NW, n_chunks, CHUNK)

    mesh = plsc.VectorSubcoreMesh(
        core_axis_name="c", subcore_axis_name="s", num_cores=NC, num_subcores=NS
    )
    out = pl.kernel(
        _gather_body,
        out_type=jax.ShapeDtypeStruct((total, D), jnp.float32),
        mesh=mesh,
        scratch_types=[
            pltpu.VMEM_SHARED(weight.shape, jnp.float32),
            pltpu.VMEM((n_chunks, CHUNK), jnp.int32),
            pltpu.VMEM((NBUF, ROWS, D), jnp.float32),
            pltpu.SemaphoreType.DMA((NBUF,)),
            pltpu.SemaphoreType.DMA((NBUF,)),
        ],
    )(x3, weight)
    return lax.stop_gradient(out.reshape(B, S, D))


# speedup vs baseline: 1.7705x; 1.0306x over previous
import jax
import jax.numpy as jnp
from jax import lax
from jax.experimental import pallas as pl
from jax.experimental.pallas import tpu as pltpu
from jax.experimental.pallas import tpu_sc as plsc

NC = 2
NS = 16
NW = NC * NS

D = 128
CHUNK = 128
GROUP = 1
ROWS = GROUP * CHUNK
NBUF = 4
TSPLIT = 5


def _gather_body(x_hbm, w_hbm, out_hbm, tbl_s, idx_v, rows_v, gsem, ssem):
    sid = lax.axis_index("s")
    wid = sid * NC + lax.axis_index("c")
    n_groups = x_hbm.shape[1] // GROUP
    base = wid * (n_groups * ROWS)

    rows_per_piece = tbl_s.shape[0] // TSPLIT

    @pl.when(sid < TSPLIT)
    def _():
        piece = pl.ds(sid * rows_per_piece, rows_per_piece)
        pltpu.async_copy(w_hbm.at[piece], tbl_s.at[piece], gsem.at[0])

    pltpu.sync_copy(x_hbm.at[wid], idx_v)

    @pl.when(sid < TSPLIT)
    def _():
        piece = pl.ds(sid * rows_per_piece, rows_per_piece)
        pltpu.make_async_copy(w_hbm.at[piece], tbl_s.at[piece], gsem.at[0]).wait()

    plsc.subcore_barrier()

    def fire_gather(j, b):
        for h in range(GROUP):
            pltpu.async_copy(
                tbl_s.at[idx_v.at[GROUP * j + h]],
                rows_v.at[b].at[pl.ds(h * CHUNK, CHUNK)],
                gsem.at[b],
            )

    def fire_scatter(j, b):
        pltpu.async_copy(
            rows_v.at[b], out_hbm.at[pl.ds(base + j * ROWS, ROWS)], ssem.at[b]
        )

    def wait_gather(b):
        pltpu.make_async_copy(
            out_hbm.at[pl.ds(base, ROWS)], rows_v.at[b], gsem.at[b]
        ).wait()

    def wait_scatter(j, b):
        pltpu.make_async_copy(
            rows_v.at[b], out_hbm.at[pl.ds(base + j * ROWS, ROWS)], ssem.at[b]
        ).wait()

    for b in range(NBUF):
        fire_gather(b, b)

    def body(o, _):
        j0 = o * NBUF
        for b in range(NBUF):
            j = j0 + b
            wait_gather(b)
            fire_scatter(j, b)
            wait_scatter(j, b)
            fire_gather(j + NBUF, b)
        return 0

    lax.fori_loop(0, (n_groups - NBUF) // NBUF, body, 0, unroll=False)

    for b in range(NBUF):
        j = n_groups - NBUF + b
        wait_gather(b)
        fire_scatter(j, b)
    for b in range(NBUF):
        wait_scatter(n_groups - NBUF + b, b)


def kernel(x, weight):
    B, S = x.shape
    total = B * S
    per_w = total // NW
    n_chunks = per_w // CHUNK
    x3 = x.reshape(NW, n_chunks, CHUNK)

    mesh = plsc.VectorSubcoreMesh(
        core_axis_name="c", subcore_axis_name="s", num_cores=NC, num_subcores=NS
    )
    out = pl.kernel(
        _gather_body,
        out_type=jax.ShapeDtypeStruct((total, D), jnp.float32),
        mesh=mesh,
        scratch_types=[
            pltpu.VMEM_SHARED(weight.shape, jnp.float32),
            pltpu.VMEM((n_chunks, CHUNK), jnp.int32),
            pltpu.VMEM((NBUF, ROWS, D), jnp.float32),
            pltpu.SemaphoreType.DMA((NBUF,)),
            pltpu.SemaphoreType.DMA((NBUF,)),
        ],
    )(x3, weight)
    return lax.stop_gradient(out.reshape(B, S, D))
